# baseline (speedup 1.0000x reference)
import functools
import jax
import jax.numpy as jnp
from jax import lax
from jax.experimental import pallas as pl
from jax.experimental.pallas import tpu as pltpu
from jax.experimental.pallas import tpu_sc as plsc

_B, _TAU, _NX, _NY, _D = 4, 16, 24, 24, 768
_D3 = 256
_GW = _NY * _D
_NGROUPS = _B * _TAU * _NX
_NW = 32
_GPW = _NGROUPS // _NW


def _sc_body(tok_hbm, x_hbm, y_hbm, t_hbm, out_hbm, xv, yv, tv, buf):
    cid = lax.axis_index("c")
    sid = lax.axis_index("s")
    wid = sid * 2 + cid
    pltpu.sync_copy(x_hbm, xv)
    pltpu.sync_copy(y_hbm, yv)
    pltpu.sync_copy(t_hbm, tv)
    g0 = wid * _GPW

    def group_body(k, carry):
        g = g0 + k
        base = g * _GW
        ti = (g // _NX) % _TAU
        ii = g % _NX
        pltpu.sync_copy(tok_hbm.at[pl.ds(base, _GW)], buf)

        def row_body(j, c2):
            row = j * _D
            for cch in range(16):
                off = cch * 16
                plsc.addupdate(buf.at[pl.ds(row + off, 16)],
                               xv[pl.ds(ii * _D3 + off, 16)])
            for cch in range(16):
                off = cch * 16
                plsc.addupdate(buf.at[pl.ds(row + _D3 + off, 16)],
                               yv[pl.ds(j * _D3 + off, 16)])
            for cch in range(16):
                off = cch * 16
                plsc.addupdate(buf.at[pl.ds(row + 2 * _D3 + off, 16)],
                               tv[pl.ds(ti * _D3 + off, 16)])
            return c2

        lax.fori_loop(0, _NY, row_body, 0)
        pltpu.sync_copy(buf, out_hbm.at[pl.ds(base, _GW)])
        return carry

    lax.fori_loop(0, _GPW, group_body, 0)


def kernel(tokens, n_x, n_y, x_emb, y_emb, t_emb):
    B, tau, N, d = tokens.shape
    tok_flat = tokens.reshape(-1)

    sc_call = functools.partial(
        pl.kernel,
        mesh=plsc.VectorSubcoreMesh(core_axis_name="c", subcore_axis_name="s"),
        out_type=jax.ShapeDtypeStruct((tok_flat.shape[0],), jnp.float32),
        scratch_types=[
            pltpu.VMEM((_NX * _D3,), jnp.float32),
            pltpu.VMEM((_NY * _D3,), jnp.float32),
            pltpu.VMEM((_TAU * _D3,), jnp.float32),
            pltpu.VMEM((_GW,), jnp.float32),
        ],
    )(_sc_body)

    out_flat = sc_call(tok_flat, x_emb.reshape(-1), y_emb.reshape(-1),
                       t_emb.reshape(-1))
    return out_flat.reshape(B, tau, N, d)


# speedup vs baseline: 7.7431x; 7.7431x over previous
import jax
import jax.numpy as jnp
from jax.experimental import pallas as pl
from jax.experimental.pallas import tpu as pltpu

_D_MODEL = 768
_D3 = _D_MODEL // 3
_TAU_BLK = 8


def _embed_add_kernel(tok_ref, x_ref, y_ref, t_ref, out_ref):
    x = x_ref[...]
    y = y_ref[...]
    d = _D3
    for i in range(_TAU_BLK):
        tok = tok_ref[0, i]
        out_ref[0, i, :, :, 0:d] = tok[:, :, 0:d] + x[:, None, :]
        out_ref[0, i, :, :, d:2 * d] = tok[:, :, d:2 * d] + y[None, :, :]
        out_ref[0, i, :, :, 2 * d:3 * d] = tok[:, :, 2 * d:3 * d] + t_ref[i]


def kernel(tokens, n_x, n_y, x_emb, y_emb, t_emb):
    B, tau, N, d = tokens.shape
    nx = x_emb.shape[0]
    ny = y_emb.shape[0]
    tok5 = tokens.reshape(B, tau, nx, ny, d)

    out5 = pl.pallas_call(
        _embed_add_kernel,
        grid=(B * tau // _TAU_BLK,),
        in_specs=[
            pl.BlockSpec((1, _TAU_BLK, nx, ny, d), lambda g: (g // 2, g % 2, 0, 0, 0)),
            pl.BlockSpec((nx, _D3), lambda g: (0, 0)),
            pl.BlockSpec((ny, _D3), lambda g: (0, 0)),
            pl.BlockSpec((_TAU_BLK, 1, _D3), lambda g: (g % 2, 0, 0)),
        ],
        out_specs=pl.BlockSpec((1, _TAU_BLK, nx, ny, d), lambda g: (g // 2, g % 2, 0, 0, 0)),
        out_shape=jax.ShapeDtypeStruct((B, tau, nx, ny, d), tokens.dtype),
        compiler_params=pltpu.CompilerParams(
            dimension_semantics=("parallel",),
            vmem_limit_bytes=128 * 1024 * 1024,
        ),
    )(tok5, x_emb, y_emb, t_emb.reshape(tau, 1, _D3))

    return out5.reshape(B, tau, N, d)
